# baseline (speedup 1.0000x reference)
import functools

import jax
import jax.numpy as jnp
from jax import lax
from jax.experimental import pallas as pl
from jax.experimental.pallas import tpu as pltpu
from jax.experimental.pallas import tpu_sc as plsc

N = 10000
NP = 10240
E = 320000
D = 128
G = 16
NC = 2
NS = 16
NW = NC * NS
EPT = E // NW
CH = 128
EPT_P = 10240
E_P = NW * EPT_P
NH = 2
HIT = EPT_P // CH // NH
RPT = NP // NS
DEGC = 128
BR = 1024
NB = NP // BR

_MESH = plsc.VectorSubcoreMesh(core_axis_name="c", subcore_axis_name="s")


@functools.partial(
    pl.kernel,
    mesh=_MESH,
    out_type=jax.ShapeDtypeStruct((NC, NP, DEGC), jnp.float32),
    scratch_types=[
        pltpu.VMEM((NH, HIT, CH), jnp.int32),
        pltpu.VMEM((CH, DEGC), jnp.float32),
        pltpu.VMEM_SHARED((NP, DEGC), jnp.float32),
    ],
)
def _sc_degree(dst_hbm, zeros_hbm, ones_hbm, out_hbm, dst_v, ones_v, acc_sh):
    cid = lax.axis_index("c")
    sid = lax.axis_index("s")
    wid = sid * NC + cid
    pltpu.sync_copy(zeros_hbm, acc_sh.at[pl.ds(sid * RPT, RPT)])
    pltpu.sync_copy(dst_hbm.at[wid], dst_v)
    pltpu.sync_copy(ones_hbm, ones_v)
    plsc.subcore_barrier()

    def body(i, carry):
        h = i // HIT
        k = i % HIT
        pltpu.sync_copy(ones_v, acc_sh.at[dst_v.at[h, k]], add=True)
        return carry

    lax.fori_loop(0, NH * HIT, body, 0)
    plsc.subcore_barrier()
    pltpu.sync_copy(acc_sh.at[pl.ds(sid * RPT, RPT)],
                    out_hbm.at[cid, pl.ds(sid * RPT, RPT)])


@functools.partial(
    pl.kernel,
    mesh=_MESH,
    out_type=jax.ShapeDtypeStruct((NC, NP, D), jnp.float32),
    scratch_types=[
        pltpu.VMEM((HIT, CH), jnp.int32),
        pltpu.VMEM((HIT, CH), jnp.int32),
        pltpu.VMEM((2, CH, D), jnp.float32),
        pltpu.VMEM_SHARED((NP, D), jnp.float32),
        pltpu.SemaphoreType.DMA,
        pltpu.SemaphoreType.DMA,
    ],
)
def _sc_aggregate(y_hbm, src_hbm, dst_hbm, zeros_hbm, out_hbm,
                  src_v, dst_v, rows_v, acc_sh, sem0, sem1):
    cid = lax.axis_index("c")
    sid = lax.axis_index("s")
    wid = sid * NC + cid
    pltpu.sync_copy(zeros_hbm, acc_sh.at[pl.ds(sid * RPT, RPT)])
    plsc.subcore_barrier()

    sems = (sem0, sem1)
    for h in range(NH):
        pltpu.sync_copy(src_hbm.at[wid, h], src_v)
        pltpu.sync_copy(dst_hbm.at[wid, h], dst_v)
        pltpu.async_copy(y_hbm.at[src_v.at[0]], rows_v.at[0], sem0)
        pltpu.async_copy(y_hbm.at[src_v.at[1]], rows_v.at[1], sem1)

        def body(j, carry):
            for b in range(2):
                i = j * 2 + b
                pltpu.make_async_copy(
                    y_hbm.at[src_v.at[i]], rows_v.at[b], sems[b]).wait()
                pltpu.sync_copy(rows_v.at[b], acc_sh.at[dst_v.at[i]],
                                add=True)
                nxt = i + 2

                @pl.when(nxt < HIT)
                def _():
                    pltpu.async_copy(y_hbm.at[src_v.at[nxt]], rows_v.at[b],
                                     sems[b])
            return carry

        lax.fori_loop(0, HIT // 2, body, 0)
    plsc.subcore_barrier()
    pltpu.sync_copy(acc_sh.at[pl.ds(sid * RPT, RPT)],
                    out_hbm.at[cid, pl.ds(sid * RPT, RPT)])


def _dinv_from(deg_ref):
    deg = deg_ref[0, :, 0] + deg_ref[1, :, 0]
    return jnp.where(deg > 0, lax.rsqrt(deg), 0.0)


def _tc1_body(x_ref, w_ref, deg_ref, o_ref):
    dinv = _dinv_from(deg_ref)
    xw = jnp.dot(x_ref[...], w_ref[...], preferred_element_type=jnp.float32)
    o_ref[...] = xw * dinv[:, None]


def _tc2_body(p_ref, deg_ref, b_ref, w_ref, o_ref):
    dinv = _dinv_from(deg_ref)
    h = (p_ref[0] + p_ref[1]) * dinv[:, None] + b_ref[...]
    hw = jnp.dot(h, w_ref[...], preferred_element_type=jnp.float32)
    o_ref[...] = hw * dinv[:, None]


def _tc3_body(p_ref, deg_ref, b_ref, batch_ref, o_ref):
    i = pl.program_id(0)
    dinv = _dinv_from(deg_ref)
    h = (p_ref[0] + p_ref[1]) * dinv[:, None] + b_ref[...]
    bvec = batch_ref[0, 0, :]
    onehot = (bvec[None, :] ==
              lax.broadcasted_iota(jnp.int32, (G, BR), 0)).astype(jnp.float32)
    contrib = jnp.dot(onehot, h, preferred_element_type=jnp.float32)

    @pl.when(i == 0)
    def _():
        o_ref[...] = jnp.zeros_like(o_ref)

    o_ref[...] += contrib


def _tc_scale_matmul(x, W, degp):
    return pl.pallas_call(
        _tc1_body,
        grid=(NB,),
        in_specs=[
            pl.BlockSpec((BR, D), lambda i: (i, 0)),
            pl.BlockSpec((D, D), lambda i: (0, 0)),
            pl.BlockSpec((NC, BR, DEGC), lambda i: (0, i, 0)),
        ],
        out_specs=pl.BlockSpec((BR, D), lambda i: (i, 0)),
        out_shape=jax.ShapeDtypeStruct((NP, D), jnp.float32),
    )(x, W, degp)


def _tc_combine_matmul(p, degp, b, W):
    return pl.pallas_call(
        _tc2_body,
        grid=(NB,),
        in_specs=[
            pl.BlockSpec((NC, BR, D), lambda i: (0, i, 0)),
            pl.BlockSpec((NC, BR, DEGC), lambda i: (0, i, 0)),
            pl.BlockSpec((1, D), lambda i: (0, 0)),
            pl.BlockSpec((D, D), lambda i: (0, 0)),
        ],
        out_specs=pl.BlockSpec((BR, D), lambda i: (i, 0)),
        out_shape=jax.ShapeDtypeStruct((NP, D), jnp.float32),
    )(p, degp, b, W)


def _tc_combine_pool(p, degp, b, batch_r):
    return pl.pallas_call(
        _tc3_body,
        grid=(NB,),
        in_specs=[
            pl.BlockSpec((NC, BR, D), lambda i: (0, i, 0)),
            pl.BlockSpec((NC, BR, DEGC), lambda i: (0, i, 0)),
            pl.BlockSpec((1, D), lambda i: (0, 0)),
            pl.BlockSpec((1, 1, BR), lambda i: (i, 0, 0)),
        ],
        out_specs=pl.BlockSpec((G, D), lambda i: (0, 0)),
        out_shape=jax.ShapeDtypeStruct((G, D), jnp.float32),
    )(p, degp, b, batch_r)


def kernel(x, edge_index, batch, W1, b1, W2, b2):
    ei = edge_index.astype(jnp.int32)
    pad = jnp.full((2, E_P - E), NP - 1, jnp.int32)
    ep = jnp.concatenate([ei, pad], axis=1)
    src = ep[0].reshape(NW, NH, HIT, CH)
    dst = ep[1].reshape(NW, NH, HIT, CH)
    xp = jnp.concatenate([x, jnp.zeros((NP - N, D), jnp.float32)], axis=0)
    batch_p = jnp.concatenate(
        [batch.astype(jnp.int32), jnp.full((NP - N,), G, jnp.int32)])
    batch_r = batch_p.reshape(NB, 1, BR)
    zeros_deg = jnp.zeros((RPT, DEGC), jnp.float32)
    ones_deg = jnp.ones((CH, DEGC), jnp.float32)
    zeros_rows = jnp.zeros((RPT, D), jnp.float32)
    b1r = b1.reshape(1, D)
    b2r = b2.reshape(1, D)

    degp = _sc_degree(dst, zeros_deg, ones_deg)
    y1 = _tc_scale_matmul(xp, W1, degp)
    p1 = _sc_aggregate(y1, src, dst, zeros_rows)
    y2 = _tc_combine_matmul(p1, degp, b1r, W2)
    p2 = _sc_aggregate(y2, src, dst, zeros_rows)
    return _tc_combine_pool(p2, degp, b2r, batch_r)


# speedup vs baseline: 1.0000x; 1.0000x over previous
import functools

import jax
import jax.numpy as jnp
from jax import lax
from jax.experimental import pallas as pl
from jax.experimental.pallas import tpu as pltpu
from jax.experimental.pallas import tpu_sc as plsc

N = 10000
NP = 10240
E = 320000
D = 128
G = 16
NC = 2
NS = 16
NW = NC * NS
EPT = E // NW
CH = 128
EPT_P = 10240
E_P = NW * EPT_P
NH = 2
HIT = EPT_P // CH // NH
RPT = NP // NS
DEGC = 128
BR = 1024
NB = NP // BR

_MESH = plsc.VectorSubcoreMesh(core_axis_name="c", subcore_axis_name="s")


@functools.partial(
    pl.kernel,
    mesh=_MESH,
    out_type=jax.ShapeDtypeStruct((NC, NP, DEGC), jnp.float32),
    scratch_types=[
        pltpu.VMEM((NH, HIT, CH), jnp.int32),
        pltpu.VMEM((CH, DEGC), jnp.float32),
        pltpu.VMEM_SHARED((NP, DEGC), jnp.float32),
    ],
)
def _sc_degree(dst_hbm, zeros_hbm, ones_hbm, out_hbm, dst_v, ones_v, acc_sh):
    cid = lax.axis_index("c")
    sid = lax.axis_index("s")
    wid = sid * NC + cid
    pltpu.sync_copy(zeros_hbm, acc_sh.at[pl.ds(sid * RPT, RPT)])
    pltpu.sync_copy(dst_hbm.at[wid], dst_v)
    pltpu.sync_copy(ones_hbm, ones_v)
    plsc.subcore_barrier()

    def body(i, carry):
        h = i // HIT
        k = i % HIT
        pltpu.sync_copy(ones_v, acc_sh.at[dst_v.at[h, k]], add=True)
        return carry

    lax.fori_loop(0, NH * HIT, body, 0)
    plsc.subcore_barrier()
    pltpu.sync_copy(acc_sh.at[pl.ds(sid * RPT, RPT)],
                    out_hbm.at[cid, pl.ds(sid * RPT, RPT)])


@functools.partial(
    pl.kernel,
    mesh=_MESH,
    out_type=jax.ShapeDtypeStruct((NC, NP, D), jnp.float32),
    scratch_types=[
        pltpu.VMEM((HIT, CH), jnp.int32),
        pltpu.VMEM((HIT, CH), jnp.int32),
        pltpu.VMEM((2, CH, D), jnp.float32),
        pltpu.VMEM_SHARED((NP, D), jnp.float32),
        pltpu.SemaphoreType.DMA,
        pltpu.SemaphoreType.DMA,
    ],
)
def _sc_aggregate(y_hbm, src_hbm, dst_hbm, zeros_hbm, out_hbm,
                  src_v, dst_v, rows_v, acc_sh, sem0, sem1):
    cid = lax.axis_index("c")
    sid = lax.axis_index("s")
    wid = sid * NC + cid
    pltpu.sync_copy(zeros_hbm, acc_sh.at[pl.ds(sid * RPT, RPT)])
    plsc.subcore_barrier()

    sems = (sem0, sem1)
    for h in range(NH):
        pltpu.sync_copy(src_hbm.at[wid, h], src_v)
        pltpu.sync_copy(dst_hbm.at[wid, h], dst_v)
        pltpu.async_copy(y_hbm.at[src_v.at[0]], rows_v.at[0], sem0)
        pltpu.async_copy(y_hbm.at[src_v.at[1]], rows_v.at[1], sem1)

        def body(j, carry):
            for b in range(2):
                i = j * 2 + b
                pltpu.make_async_copy(
                    y_hbm.at[src_v.at[i]], rows_v.at[b], sems[b]).wait()
                pltpu.sync_copy(rows_v.at[b], acc_sh.at[dst_v.at[i]],
                                add=True)
                pltpu.async_copy(y_hbm.at[src_v.at[i + 2]], rows_v.at[b],
                                 sems[b])
            return carry

        lax.fori_loop(0, HIT // 2 - 1, body, 0)
        for b in range(2):
            i = HIT - 2 + b
            pltpu.make_async_copy(
                y_hbm.at[src_v.at[i]], rows_v.at[b], sems[b]).wait()
            pltpu.sync_copy(rows_v.at[b], acc_sh.at[dst_v.at[i]], add=True)
    plsc.subcore_barrier()
    pltpu.sync_copy(acc_sh.at[pl.ds(sid * RPT, RPT)],
                    out_hbm.at[cid, pl.ds(sid * RPT, RPT)])


def _dinv_from(deg_ref):
    deg = deg_ref[0, :, 0] + deg_ref[1, :, 0]
    return jnp.where(deg > 0, lax.rsqrt(deg), 0.0)


def _tc1_body(x_ref, w_ref, deg_ref, o_ref):
    dinv = _dinv_from(deg_ref)
    xw = jnp.dot(x_ref[...], w_ref[...], preferred_element_type=jnp.float32)
    o_ref[...] = xw * dinv[:, None]


def _tc2_body(p_ref, deg_ref, b_ref, w_ref, o_ref):
    dinv = _dinv_from(deg_ref)
    h = (p_ref[0] + p_ref[1]) * dinv[:, None] + b_ref[...]
    hw = jnp.dot(h, w_ref[...], preferred_element_type=jnp.float32)
    o_ref[...] = hw * dinv[:, None]


def _tc3_body(p_ref, deg_ref, b_ref, batch_ref, o_ref):
    i = pl.program_id(0)
    dinv = _dinv_from(deg_ref)
    h = (p_ref[0] + p_ref[1]) * dinv[:, None] + b_ref[...]
    bvec = batch_ref[0, 0, :]
    onehot = (bvec[None, :] ==
              lax.broadcasted_iota(jnp.int32, (G, BR), 0)).astype(jnp.float32)
    contrib = jnp.dot(onehot, h, preferred_element_type=jnp.float32)

    @pl.when(i == 0)
    def _():
        o_ref[...] = jnp.zeros_like(o_ref)

    o_ref[...] += contrib


def _tc_scale_matmul(x, W, degp):
    return pl.pallas_call(
        _tc1_body,
        grid=(NB,),
        in_specs=[
            pl.BlockSpec((BR, D), lambda i: (i, 0)),
            pl.BlockSpec((D, D), lambda i: (0, 0)),
            pl.BlockSpec((NC, BR, DEGC), lambda i: (0, i, 0)),
        ],
        out_specs=pl.BlockSpec((BR, D), lambda i: (i, 0)),
        out_shape=jax.ShapeDtypeStruct((NP, D), jnp.float32),
    )(x, W, degp)


def _tc_combine_matmul(p, degp, b, W):
    return pl.pallas_call(
        _tc2_body,
        grid=(NB,),
        in_specs=[
            pl.BlockSpec((NC, BR, D), lambda i: (0, i, 0)),
            pl.BlockSpec((NC, BR, DEGC), lambda i: (0, i, 0)),
            pl.BlockSpec((1, D), lambda i: (0, 0)),
            pl.BlockSpec((D, D), lambda i: (0, 0)),
        ],
        out_specs=pl.BlockSpec((BR, D), lambda i: (i, 0)),
        out_shape=jax.ShapeDtypeStruct((NP, D), jnp.float32),
    )(p, degp, b, W)


def _tc_combine_pool(p, degp, b, batch_r):
    return pl.pallas_call(
        _tc3_body,
        grid=(NB,),
        in_specs=[
            pl.BlockSpec((NC, BR, D), lambda i: (0, i, 0)),
            pl.BlockSpec((NC, BR, DEGC), lambda i: (0, i, 0)),
            pl.BlockSpec((1, D), lambda i: (0, 0)),
            pl.BlockSpec((1, 1, BR), lambda i: (i, 0, 0)),
        ],
        out_specs=pl.BlockSpec((G, D), lambda i: (0, 0)),
        out_shape=jax.ShapeDtypeStruct((G, D), jnp.float32),
    )(p, degp, b, batch_r)


def kernel(x, edge_index, batch, W1, b1, W2, b2):
    ei = edge_index.astype(jnp.int32)
    pad = jnp.full((2, E_P - E), NP - 1, jnp.int32)
    ep = jnp.concatenate([ei, pad], axis=1)
    src = ep[0].reshape(NW, NH, HIT, CH)
    dst = ep[1].reshape(NW, NH, HIT, CH)
    xp = jnp.concatenate([x, jnp.zeros((NP - N, D), jnp.float32)], axis=0)
    batch_p = jnp.concatenate(
        [batch.astype(jnp.int32), jnp.full((NP - N,), G, jnp.int32)])
    batch_r = batch_p.reshape(NB, 1, BR)
    zeros_deg = jnp.zeros((RPT, DEGC), jnp.float32)
    ones_deg = jnp.ones((CH, DEGC), jnp.float32)
    zeros_rows = jnp.zeros((RPT, D), jnp.float32)
    b1r = b1.reshape(1, D)
    b2r = b2.reshape(1, D)

    degp = _sc_degree(dst, zeros_deg, ones_deg)
    y1 = _tc_scale_matmul(xp, W1, degp)
    p1 = _sc_aggregate(y1, src, dst, zeros_rows)
    y2 = _tc_combine_matmul(p1, degp, b1r, W2)
    p2 = _sc_aggregate(y2, src, dst, zeros_rows)
    return _tc_combine_pool(p2, degp, b2r, batch_r)


# speedup vs baseline: 2.9284x; 2.9283x over previous
import functools

import jax
import jax.numpy as jnp
from jax import lax
from jax.experimental import pallas as pl
from jax.experimental.pallas import tpu as pltpu
from jax.experimental.pallas import tpu_sc as plsc

N = 10000
NP = 10240
E = 320000
D = 128
G = 16
NC = 2
NS = 16
NW = NC * NS
EPT = E // NW
CH = 128
EPT_P = 10240
E_P = NW * EPT_P
NH = 2
HIT = EPT_P // CH // NH
RPT = NP // NS
DEGC = 128
BR = 1024
NB = NP // BR

_MESH = plsc.VectorSubcoreMesh(core_axis_name="c", subcore_axis_name="s")


@functools.partial(
    pl.kernel,
    mesh=_MESH,
    out_type=jax.ShapeDtypeStruct((NC, NP, DEGC), jnp.float32),
    scratch_types=[
        pltpu.VMEM((NH, HIT, CH), jnp.int32),
        pltpu.VMEM((CH, DEGC), jnp.float32),
        pltpu.VMEM_SHARED((NP, DEGC), jnp.float32),
    ],
)
def _sc_degree(dst_hbm, zeros_hbm, ones_hbm, out_hbm, dst_v, ones_v, acc_sh):
    cid = lax.axis_index("c")
    sid = lax.axis_index("s")
    wid = sid * NC + cid
    pltpu.sync_copy(zeros_hbm, acc_sh.at[pl.ds(sid * RPT, RPT)])
    pltpu.sync_copy(dst_hbm.at[wid], dst_v)
    pltpu.sync_copy(ones_hbm, ones_v)
    plsc.subcore_barrier()

    def body(i, carry):
        h = i // HIT
        k = i % HIT
        pltpu.sync_copy(ones_v, acc_sh.at[dst_v.at[h, k]], add=True)
        return carry

    lax.fori_loop(0, NH * HIT, body, 0)
    plsc.subcore_barrier()
    pltpu.sync_copy(acc_sh.at[pl.ds(sid * RPT, RPT)],
                    out_hbm.at[cid, pl.ds(sid * RPT, RPT)])


@functools.partial(
    pl.kernel,
    mesh=_MESH,
    out_type=jax.ShapeDtypeStruct((NC, NP, D), jnp.float32),
    scratch_types=[
        pltpu.VMEM((HIT, CH), jnp.int32),
        pltpu.VMEM((HIT, CH), jnp.int32),
        pltpu.VMEM((2, CH, D), jnp.float32),
        pltpu.VMEM_SHARED((NP, D), jnp.float32),
        pltpu.SemaphoreType.DMA,
        pltpu.SemaphoreType.DMA,
    ],
)
def _sc_aggregate(y_hbm, src_hbm, dst_hbm, zeros_hbm, out_hbm,
                  src_v, dst_v, rows_v, acc_sh, sem0, sem1):
    cid = lax.axis_index("c")
    sid = lax.axis_index("s")
    wid = sid * NC + cid
    pltpu.sync_copy(zeros_hbm, acc_sh.at[pl.ds(sid * RPT, RPT)])
    plsc.subcore_barrier()

    sems = (sem0, sem1)
    for h in range(NH):
        pltpu.sync_copy(src_hbm.at[wid, h], src_v)
        pltpu.sync_copy(dst_hbm.at[wid, h], dst_v)
        pltpu.async_copy(y_hbm.at[src_v.at[0]], rows_v.at[0], sem0)
        pltpu.async_copy(y_hbm.at[src_v.at[1]], rows_v.at[1], sem1)

        def body(j, carry):
            for b in range(2):
                i = j * 2 + b
                pltpu.make_async_copy(
                    y_hbm.at[src_v.at[i]], rows_v.at[b], sems[b]).wait()
                pltpu.sync_copy(rows_v.at[b], acc_sh.at[dst_v.at[i]],
                                add=True)
                pltpu.async_copy(y_hbm.at[src_v.at[i + 2]], rows_v.at[b],
                                 sems[b])
            return carry

        lax.fori_loop(0, HIT // 2 - 1, body, 0)
        for b in range(2):
            i = HIT - 2 + b
            pltpu.make_async_copy(
                y_hbm.at[src_v.at[i]], rows_v.at[b], sems[b]).wait()
            pltpu.sync_copy(rows_v.at[b], acc_sh.at[dst_v.at[i]], add=True)
    plsc.subcore_barrier()
    pltpu.sync_copy(acc_sh.at[pl.ds(sid * RPT, RPT)],
                    out_hbm.at[cid, pl.ds(sid * RPT, RPT)])


def _dinv_from(deg_ref):
    deg = deg_ref[0, :, 0] + deg_ref[1, :, 0]
    return jnp.where(deg > 0, lax.rsqrt(deg), 0.0)


def _tc1_body(x_ref, w_ref, deg_ref, o_ref):
    dinv = _dinv_from(deg_ref)
    xw = jnp.dot(x_ref[...], w_ref[...], preferred_element_type=jnp.float32)
    o_ref[...] = xw * dinv[:, None]


def _tc2_body(p_ref, deg_ref, b_ref, w_ref, o_ref):
    dinv = _dinv_from(deg_ref)
    h = (p_ref[0] + p_ref[1]) * dinv[:, None] + b_ref[...]
    hw = jnp.dot(h, w_ref[...], preferred_element_type=jnp.float32)
    o_ref[...] = hw * dinv[:, None]


def _tc3_body(p_ref, deg_ref, b_ref, batch_ref, o_ref):
    i = pl.program_id(0)
    dinv = _dinv_from(deg_ref)
    h = (p_ref[0] + p_ref[1]) * dinv[:, None] + b_ref[...]
    bvec = batch_ref[0, 0, :]
    onehot = (bvec[None, :] ==
              lax.broadcasted_iota(jnp.int32, (G, BR), 0)).astype(jnp.float32)
    contrib = jnp.dot(onehot, h, preferred_element_type=jnp.float32)

    @pl.when(i == 0)
    def _():
        o_ref[...] = jnp.zeros_like(o_ref)

    o_ref[...] += contrib


def _tc_scale_matmul(x, W, degp):
    return pl.pallas_call(
        _tc1_body,
        grid=(NB,),
        in_specs=[
            pl.BlockSpec((BR, D), lambda i: (i, 0)),
            pl.BlockSpec((D, D), lambda i: (0, 0)),
            pl.BlockSpec((NC, BR, DEGC), lambda i: (0, i, 0)),
        ],
        out_specs=pl.BlockSpec((BR, D), lambda i: (i, 0)),
        out_shape=jax.ShapeDtypeStruct((NP, D), jnp.float32),
    )(x, W, degp)


def _tc_combine_matmul(p, degp, b, W):
    return pl.pallas_call(
        _tc2_body,
        grid=(NB,),
        in_specs=[
            pl.BlockSpec((NC, BR, D), lambda i: (0, i, 0)),
            pl.BlockSpec((NC, BR, DEGC), lambda i: (0, i, 0)),
            pl.BlockSpec((1, D), lambda i: (0, 0)),
            pl.BlockSpec((D, D), lambda i: (0, 0)),
        ],
        out_specs=pl.BlockSpec((BR, D), lambda i: (i, 0)),
        out_shape=jax.ShapeDtypeStruct((NP, D), jnp.float32),
    )(p, degp, b, W)


def _tc_combine_pool(p, degp, b, batch_r):
    return pl.pallas_call(
        _tc3_body,
        grid=(NB,),
        in_specs=[
            pl.BlockSpec((NC, BR, D), lambda i: (0, i, 0)),
            pl.BlockSpec((NC, BR, DEGC), lambda i: (0, i, 0)),
            pl.BlockSpec((1, D), lambda i: (0, 0)),
            pl.BlockSpec((1, 1, BR), lambda i: (i, 0, 0)),
        ],
        out_specs=pl.BlockSpec((G, D), lambda i: (0, 0)),
        out_shape=jax.ShapeDtypeStruct((G, D), jnp.float32),
    )(p, degp, b, batch_r)


def kernel(x, edge_index, batch, W1, b1, W2, b2):
    ei = edge_index.astype(jnp.int32)
    padv = N + jnp.arange(E_P - E, dtype=jnp.int32) % (NP - N)
    ep = jnp.concatenate([ei, jnp.stack([padv, padv])], axis=1)
    src = ep[0].reshape(NW, NH, HIT, CH)
    dst = ep[1].reshape(NW, NH, HIT, CH)
    xp = jnp.concatenate([x, jnp.zeros((NP - N, D), jnp.float32)], axis=0)
    batch_p = jnp.concatenate(
        [batch.astype(jnp.int32), jnp.full((NP - N,), G, jnp.int32)])
    batch_r = batch_p.reshape(NB, 1, BR)
    zeros_deg = jnp.zeros((RPT, DEGC), jnp.float32)
    ones_deg = jnp.ones((CH, DEGC), jnp.float32)
    zeros_rows = jnp.zeros((RPT, D), jnp.float32)
    b1r = b1.reshape(1, D)
    b2r = b2.reshape(1, D)

    degp = _sc_degree(dst, zeros_deg, ones_deg)
    y1 = _tc_scale_matmul(xp, W1, degp)
    p1 = _sc_aggregate(y1, src, dst, zeros_rows)
    y2 = _tc_combine_matmul(p1, degp, b1r, W2)
    p2 = _sc_aggregate(y2, src, dst, zeros_rows)
    return _tc_combine_pool(p2, degp, b2r, batch_r)


# speedup vs baseline: 2.9417x; 1.0045x over previous
import functools

import jax
import jax.numpy as jnp
from jax import lax
from jax.experimental import pallas as pl
from jax.experimental.pallas import tpu as pltpu
from jax.experimental.pallas import tpu_sc as plsc

N = 10000
NP = 10240
E = 320000
D = 128
G = 16
NC = 2
NS = 16
NW = NC * NS
EPT = E // NW
CH = 128
EPT_P = 10240
E_P = NW * EPT_P
NH = 2
HIT = EPT_P // CH // NH
RPT = NP // NS
DEGC = 128
BR = 1024
NB = NP // BR

_MESH = plsc.VectorSubcoreMesh(core_axis_name="c", subcore_axis_name="s")


@functools.partial(
    pl.kernel,
    mesh=_MESH,
    out_type=jax.ShapeDtypeStruct((NC, NP, DEGC), jnp.float32),
    scratch_types=[
        pltpu.VMEM((NH, HIT, CH), jnp.int32),
        pltpu.VMEM((CH, DEGC), jnp.float32),
        pltpu.VMEM_SHARED((NP, DEGC), jnp.float32),
    ],
)
def _sc_degree(dst_hbm, zeros_hbm, ones_hbm, out_hbm, dst_v, ones_v, acc_sh):
    cid = lax.axis_index("c")
    sid = lax.axis_index("s")
    wid = sid * NC + cid
    pltpu.sync_copy(zeros_hbm, acc_sh.at[pl.ds(sid * RPT, RPT)])
    pltpu.sync_copy(dst_hbm.at[wid], dst_v)
    pltpu.sync_copy(ones_hbm, ones_v)
    plsc.subcore_barrier()

    def body(i, carry):
        h = i // HIT
        k = i % HIT
        pltpu.sync_copy(ones_v, acc_sh.at[dst_v.at[h, k]], add=True)
        return carry

    lax.fori_loop(0, NH * HIT, body, 0)
    plsc.subcore_barrier()
    pltpu.sync_copy(acc_sh.at[pl.ds(sid * RPT, RPT)],
                    out_hbm.at[cid, pl.ds(sid * RPT, RPT)])


@functools.partial(
    pl.kernel,
    mesh=_MESH,
    out_type=jax.ShapeDtypeStruct((NC, NP, D), jnp.float32),
    scratch_types=[
        pltpu.VMEM((HIT, CH), jnp.int32),
        pltpu.VMEM((HIT, CH), jnp.int32),
        pltpu.VMEM((2, CH, D), jnp.float32),
        pltpu.VMEM_SHARED((NP, D), jnp.float32),
        pltpu.SemaphoreType.DMA,
        pltpu.SemaphoreType.DMA,
    ],
)
def _sc_aggregate(y_hbm, src_hbm, dst_hbm, zeros_hbm, out_hbm,
                  src_v, dst_v, rows_v, acc_sh, sem0, sem1):
    cid = lax.axis_index("c")
    sid = lax.axis_index("s")
    wid = sid * NC + cid
    pltpu.sync_copy(zeros_hbm, acc_sh.at[pl.ds(sid * RPT, RPT)])
    plsc.subcore_barrier()

    sems = (sem0, sem1)
    for h in range(NH):
        pltpu.sync_copy(src_hbm.at[wid, h], src_v)
        pltpu.sync_copy(dst_hbm.at[wid, h], dst_v)
        pltpu.async_copy(y_hbm.at[src_v.at[0]], rows_v.at[0], sem0)
        pltpu.async_copy(y_hbm.at[src_v.at[1]], rows_v.at[1], sem1)

        def body(j, carry):
            for b in range(2):
                i = j * 2 + b
                pltpu.make_async_copy(
                    y_hbm.at[src_v.at[i]], rows_v.at[b], sems[b]).wait()
                pltpu.sync_copy(rows_v.at[b], acc_sh.at[dst_v.at[i]],
                                add=True)
                pltpu.async_copy(y_hbm.at[src_v.at[i + 2]], rows_v.at[b],
                                 sems[b])
            return carry

        lax.fori_loop(0, HIT // 2 - 1, body, 0)
        for b in range(2):
            i = HIT - 2 + b
            pltpu.make_async_copy(
                y_hbm.at[src_v.at[i]], rows_v.at[b], sems[b]).wait()
            pltpu.sync_copy(rows_v.at[b], acc_sh.at[dst_v.at[i]], add=True)
    plsc.subcore_barrier()
    pltpu.sync_copy(acc_sh.at[pl.ds(sid * RPT, RPT)],
                    out_hbm.at[cid, pl.ds(sid * RPT, RPT)])


def _dinv_from(deg_ref):
    deg = deg_ref[0, :, 0] + deg_ref[1, :, 0]
    return jnp.where(deg > 0, lax.rsqrt(deg), 0.0)


def _tc_mm_body(x_ref, w_ref, o_ref):
    o_ref[...] = jnp.dot(x_ref[...], w_ref[...],
                         preferred_element_type=jnp.float32)


def _tc_scale_body(xw_ref, deg_ref, o_ref, dinv_ref):
    dinv = _dinv_from(deg_ref)
    o_ref[...] = xw_ref[...] * dinv[:, None]
    dinv_ref[0, 0, :] = dinv


def _tc2_body(p_ref, dinv_ref, b_ref, w_ref, o_ref):
    dinv = dinv_ref[0, 0, :]
    h = (p_ref[0] + p_ref[1]) * dinv[:, None] + b_ref[...]
    hw = jnp.dot(h, w_ref[...], preferred_element_type=jnp.float32)
    o_ref[...] = hw * dinv[:, None]


def _tc3_body(p_ref, dinv_ref, b_ref, batch_ref, o_ref):
    i = pl.program_id(0)
    dinv = dinv_ref[0, 0, :]
    h = (p_ref[0] + p_ref[1]) * dinv[:, None] + b_ref[...]
    bvec = batch_ref[0, 0, :]
    onehot = (bvec[None, :] ==
              lax.broadcasted_iota(jnp.int32, (G, BR), 0)).astype(jnp.float32)
    contrib = jnp.dot(onehot, h, preferred_element_type=jnp.float32)

    @pl.when(i == 0)
    def _():
        o_ref[...] = jnp.zeros_like(o_ref)

    o_ref[...] += contrib


def _tc_matmul(x, W):
    return pl.pallas_call(
        _tc_mm_body,
        grid=(NB,),
        in_specs=[
            pl.BlockSpec((BR, D), lambda i: (i, 0)),
            pl.BlockSpec((D, D), lambda i: (0, 0)),
        ],
        out_specs=pl.BlockSpec((BR, D), lambda i: (i, 0)),
        out_shape=jax.ShapeDtypeStruct((NP, D), jnp.float32),
    )(x, W)


def _tc_scale(xw, degp):
    return pl.pallas_call(
        _tc_scale_body,
        grid=(NB,),
        in_specs=[
            pl.BlockSpec((BR, D), lambda i: (i, 0)),
            pl.BlockSpec((NC, BR, DEGC), lambda i: (0, i, 0)),
        ],
        out_specs=[
            pl.BlockSpec((BR, D), lambda i: (i, 0)),
            pl.BlockSpec((1, 1, BR), lambda i: (i, 0, 0)),
        ],
        out_shape=[
            jax.ShapeDtypeStruct((NP, D), jnp.float32),
            jax.ShapeDtypeStruct((NB, 1, BR), jnp.float32),
        ],
    )(xw, degp)


def _tc_combine_matmul(p, dinv3, b, W):
    return pl.pallas_call(
        _tc2_body,
        grid=(NB,),
        in_specs=[
            pl.BlockSpec((NC, BR, D), lambda i: (0, i, 0)),
            pl.BlockSpec((1, 1, BR), lambda i: (i, 0, 0)),
            pl.BlockSpec((1, D), lambda i: (0, 0)),
            pl.BlockSpec((D, D), lambda i: (0, 0)),
        ],
        out_specs=pl.BlockSpec((BR, D), lambda i: (i, 0)),
        out_shape=jax.ShapeDtypeStruct((NP, D), jnp.float32),
    )(p, dinv3, b, W)


def _tc_combine_pool(p, dinv3, b, batch_r):
    return pl.pallas_call(
        _tc3_body,
        grid=(NB,),
        in_specs=[
            pl.BlockSpec((NC, BR, D), lambda i: (0, i, 0)),
            pl.BlockSpec((1, 1, BR), lambda i: (i, 0, 0)),
            pl.BlockSpec((1, D), lambda i: (0, 0)),
            pl.BlockSpec((1, 1, BR), lambda i: (i, 0, 0)),
        ],
        out_specs=pl.BlockSpec((G, D), lambda i: (0, 0)),
        out_shape=jax.ShapeDtypeStruct((G, D), jnp.float32),
    )(p, dinv3, b, batch_r)


def kernel(x, edge_index, batch, W1, b1, W2, b2):
    ei = edge_index.astype(jnp.int32)
    padv = N + jnp.arange(E_P - E, dtype=jnp.int32) % (NP - N)
    ep = jnp.concatenate([ei, jnp.stack([padv, padv])], axis=1)
    src = ep[0].reshape(NW, NH, HIT, CH)
    dst = ep[1].reshape(NW, NH, HIT, CH)
    xp = jnp.concatenate([x, jnp.zeros((NP - N, D), jnp.float32)], axis=0)
    batch_p = jnp.concatenate(
        [batch.astype(jnp.int32), jnp.full((NP - N,), G, jnp.int32)])
    batch_r = batch_p.reshape(NB, 1, BR)
    zeros_deg = jnp.zeros((RPT, DEGC), jnp.float32)
    ones_deg = jnp.ones((CH, DEGC), jnp.float32)
    zeros_rows = jnp.zeros((RPT, D), jnp.float32)
    b1r = b1.reshape(1, D)
    b2r = b2.reshape(1, D)

    xw1 = _tc_matmul(xp, W1)
    degp = _sc_degree(dst, zeros_deg, ones_deg)
    y1, dinv3 = _tc_scale(xw1, degp)
    p1 = _sc_aggregate(y1, src, dst, zeros_rows)
    y2 = _tc_combine_matmul(p1, dinv3, b1r, W2)
    p2 = _sc_aggregate(y2, src, dst, zeros_rows)
    return _tc_combine_pool(p2, dinv3, b2r, batch_r)


# speedup vs baseline: 2.9519x; 1.0035x over previous
import functools

import jax
import jax.numpy as jnp
from jax import lax
from jax.experimental import pallas as pl
from jax.experimental.pallas import tpu as pltpu
from jax.experimental.pallas import tpu_sc as plsc

N = 10000
NP = 10240
E = 320000
D = 128
G = 16
NC = 2
NS = 16
NW = NC * NS
EPT = E // NW
CH = 128
EPT_P = 10240
E_P = NW * EPT_P
NH = 2
HIT = EPT_P // CH // NH
RPT = NP // NS
DEGC = 128
BR = 1024
NB = NP // BR

_MESH = plsc.VectorSubcoreMesh(core_axis_name="c", subcore_axis_name="s")


@functools.partial(
    pl.kernel,
    mesh=_MESH,
    out_type=jax.ShapeDtypeStruct((NC, NP, DEGC), jnp.float32),
    scratch_types=[
        pltpu.VMEM((NH, HIT, CH), jnp.int32),
        pltpu.VMEM((CH, DEGC), jnp.float32),
        pltpu.VMEM_SHARED((NP, DEGC), jnp.float32),
    ],
)
def _sc_degree(dst_hbm, zeros_hbm, ones_hbm, out_hbm, dst_v, ones_v, acc_sh):
    cid = lax.axis_index("c")
    sid = lax.axis_index("s")
    wid = sid * NC + cid
    pltpu.sync_copy(zeros_hbm, acc_sh.at[pl.ds(sid * RPT, RPT)])
    pltpu.sync_copy(dst_hbm.at[wid], dst_v)
    pltpu.sync_copy(ones_hbm, ones_v)
    plsc.subcore_barrier()

    def body(i, carry):
        h = i // HIT
        k = i % HIT
        pltpu.sync_copy(ones_v, acc_sh.at[dst_v.at[h, k]], add=True)
        return carry

    lax.fori_loop(0, NH * HIT, body, 0)
    plsc.subcore_barrier()
    pltpu.sync_copy(acc_sh.at[pl.ds(sid * RPT, RPT)],
                    out_hbm.at[cid, pl.ds(sid * RPT, RPT)])


@functools.partial(
    pl.kernel,
    mesh=_MESH,
    out_type=jax.ShapeDtypeStruct((NC, NP, D), jnp.float32),
    scratch_types=[
        pltpu.VMEM((HIT, CH), jnp.int32),
        pltpu.VMEM((HIT, CH), jnp.int32),
        pltpu.VMEM((2, CH, D), jnp.float32),
        pltpu.VMEM_SHARED((NP, D), jnp.float32),
        pltpu.SemaphoreType.DMA,
        pltpu.SemaphoreType.DMA,
    ],
)
def _sc_aggregate(y_hbm, src_hbm, dst_hbm, zeros_hbm, out_hbm,
                  src_v, dst_v, rows_v, acc_sh, sem0, sem1):
    cid = lax.axis_index("c")
    sid = lax.axis_index("s")
    wid = sid * NC + cid
    pltpu.sync_copy(zeros_hbm, acc_sh.at[pl.ds(sid * RPT, RPT)])
    plsc.subcore_barrier()

    sems = (sem0, sem1)
    for h in range(NH):
        pltpu.sync_copy(src_hbm.at[wid, h], src_v)
        pltpu.sync_copy(dst_hbm.at[wid, h], dst_v)
        pltpu.async_copy(y_hbm.at[src_v.at[0]], rows_v.at[0], sem0)
        pltpu.async_copy(y_hbm.at[src_v.at[1]], rows_v.at[1], sem1)

        def body(j, carry):
            for b in range(2):
                i = j * 2 + b
                pltpu.make_async_copy(
                    y_hbm.at[src_v.at[i]], rows_v.at[b], sems[b]).wait()
                pltpu.sync_copy(rows_v.at[b], acc_sh.at[dst_v.at[i]],
                                add=True)
                pltpu.async_copy(y_hbm.at[src_v.at[i + 2]], rows_v.at[b],
                                 sems[b])
            return carry

        lax.fori_loop(0, HIT // 2 - 1, body, 0)
        for b in range(2):
            i = HIT - 2 + b
            pltpu.make_async_copy(
                y_hbm.at[src_v.at[i]], rows_v.at[b], sems[b]).wait()
            pltpu.sync_copy(rows_v.at[b], acc_sh.at[dst_v.at[i]], add=True)
    plsc.subcore_barrier()
    pltpu.sync_copy(acc_sh.at[pl.ds(sid * RPT, RPT)],
                    out_hbm.at[cid, pl.ds(sid * RPT, RPT)])


def _dinv_from(deg_ref):
    deg = deg_ref[0, :, 0] + deg_ref[1, :, 0]
    return jnp.where(deg > 0, lax.rsqrt(deg), 0.0)


def _tc1_body(x_ref, w_ref, deg_ref, o_ref, dinv_ref):
    dinv = _dinv_from(deg_ref)
    xw = jnp.dot(x_ref[...], w_ref[...], preferred_element_type=jnp.float32)
    o_ref[...] = xw * dinv[:, None]
    dinv_ref[0, 0, :] = dinv


def _tc2_body(p_ref, dinv_ref, b_ref, w_ref, o_ref):
    dinv = dinv_ref[0, 0, :]
    h = (p_ref[0] + p_ref[1]) * dinv[:, None] + b_ref[...]
    hw = jnp.dot(h, w_ref[...], preferred_element_type=jnp.float32)
    o_ref[...] = hw * dinv[:, None]


def _tc3_body(p_ref, dinv_ref, b_ref, batch_ref, o_ref):
    i = pl.program_id(0)
    dinv = dinv_ref[0, 0, :]
    h = (p_ref[0] + p_ref[1]) * dinv[:, None] + b_ref[...]
    bvec = batch_ref[0, 0, :]
    onehot = (bvec[None, :] ==
              lax.broadcasted_iota(jnp.int32, (G, BR), 0)).astype(jnp.float32)
    contrib = jnp.dot(onehot, h, preferred_element_type=jnp.float32)

    @pl.when(i == 0)
    def _():
        o_ref[...] = jnp.zeros_like(o_ref)

    o_ref[...] += contrib


def _tc_scale_matmul(x, W, degp):
    return pl.pallas_call(
        _tc1_body,
        grid=(NB,),
        in_specs=[
            pl.BlockSpec((BR, D), lambda i: (i, 0)),
            pl.BlockSpec((D, D), lambda i: (0, 0)),
            pl.BlockSpec((NC, BR, DEGC), lambda i: (0, i, 0)),
        ],
        out_specs=[
            pl.BlockSpec((BR, D), lambda i: (i, 0)),
            pl.BlockSpec((1, 1, BR), lambda i: (i, 0, 0)),
        ],
        out_shape=[
            jax.ShapeDtypeStruct((NP, D), jnp.float32),
            jax.ShapeDtypeStruct((NB, 1, BR), jnp.float32),
        ],
    )(x, W, degp)


def _tc_combine_matmul(p, dinv3, b, W):
    return pl.pallas_call(
        _tc2_body,
        grid=(NB,),
        in_specs=[
            pl.BlockSpec((NC, BR, D), lambda i: (0, i, 0)),
            pl.BlockSpec((1, 1, BR), lambda i: (i, 0, 0)),
            pl.BlockSpec((1, D), lambda i: (0, 0)),
            pl.BlockSpec((D, D), lambda i: (0, 0)),
        ],
        out_specs=pl.BlockSpec((BR, D), lambda i: (i, 0)),
        out_shape=jax.ShapeDtypeStruct((NP, D), jnp.float32),
    )(p, dinv3, b, W)


def _tc_combine_pool(p, dinv3, b, batch_r):
    return pl.pallas_call(
        _tc3_body,
        grid=(NB,),
        in_specs=[
            pl.BlockSpec((NC, BR, D), lambda i: (0, i, 0)),
            pl.BlockSpec((1, 1, BR), lambda i: (i, 0, 0)),
            pl.BlockSpec((1, D), lambda i: (0, 0)),
            pl.BlockSpec((1, 1, BR), lambda i: (i, 0, 0)),
        ],
        out_specs=pl.BlockSpec((G, D), lambda i: (0, 0)),
        out_shape=jax.ShapeDtypeStruct((G, D), jnp.float32),
    )(p, dinv3, b, batch_r)


def kernel(x, edge_index, batch, W1, b1, W2, b2):
    ei = edge_index.astype(jnp.int32)
    padv = N + jnp.arange(E_P - E, dtype=jnp.int32) % (NP - N)
    ep = jnp.concatenate([ei, jnp.stack([padv, padv])], axis=1)
    src = ep[0].reshape(NW, NH, HIT, CH)
    dst = ep[1].reshape(NW, NH, HIT, CH)
    xp = jnp.concatenate([x, jnp.zeros((NP - N, D), jnp.float32)], axis=0)
    batch_p = jnp.concatenate(
        [batch.astype(jnp.int32), jnp.full((NP - N,), G, jnp.int32)])
    batch_r = batch_p.reshape(NB, 1, BR)
    zeros_deg = jnp.zeros((RPT, DEGC), jnp.float32)
    ones_deg = jnp.ones((CH, DEGC), jnp.float32)
    zeros_rows = jnp.zeros((RPT, D), jnp.float32)
    b1r = b1.reshape(1, D)
    b2r = b2.reshape(1, D)

    degp = _sc_degree(dst, zeros_deg, ones_deg)
    y1, dinv3 = _tc_scale_matmul(xp, W1, degp)
    p1 = _sc_aggregate(y1, src, dst, zeros_rows)
    y2 = _tc_combine_matmul(p1, dinv3, b1r, W2)
    p2 = _sc_aggregate(y2, src, dst, zeros_rows)
    return _tc_combine_pool(p2, dinv3, b2r, batch_r)
